# baseline (speedup 1.0000x reference)
import functools

import numpy as np
import jax
import jax.numpy as jnp
from jax import lax
from jax.experimental import pallas as pl
from jax.experimental.pallas import tpu as pltpu
from jax.experimental.pallas import tpu_sc as plsc

SIGMA = 25.0
NCLS = 10
HID = 64
NW = 32
BN = 1000
BE = 1024
CH = 320
NPAD = 10240


def _prep_body(x_ref, grp_ref, cat_ref, t_ref, wgfp_ref, emb_ref, wec_ref,
               bec_ref, wi1_ref, bi1_ref, wi2_ref, bi2_ref, wel_ref, bel_ref,
               wab_ref, bab_ref, a_ref, b_ref, xs_ref, rcp_ref):
    bs = t_ref.shape[0]
    t0 = t_ref[...]
    xproj = t0 * wgfp_ref[...] * np.float32(2.0 * np.pi)
    gfp = jnp.concatenate([jnp.sin(xproj), jnp.cos(xproj)], axis=1)
    xs200 = jnp.maximum(
        jnp.maximum(gfp, 0.0) @ wel_ref[...] + bel_ref[...], 0.0)
    lns = np.float32(np.log(SIGMA))
    std200 = jnp.sqrt((jnp.exp(2.0 * lns * t0) - 1.0) / (2.0 * lns))
    rcp200 = 1.0 / (std200 + np.float32(1e-7))

    grp = grp_ref[...]
    oh = (grp == lax.broadcasted_iota(jnp.int32, (BN, bs), 1)
          ).astype(jnp.float32)
    xs_ref[...] = oh @ xs200
    rcp_ref[...] = oh @ rcp200

    cat = cat_ref[...]
    oh10 = (cat == lax.broadcasted_iota(jnp.int32, (BN, NCLS), 1)
            ).astype(jnp.float32)
    cf = jnp.maximum(oh10 @ emb_ref[...], 0.0) @ wec_ref[...] + bec_ref[...]

    i1 = jnp.maximum(x_ref[...] @ wi1_ref[...] + bi1_ref[...], 0.0)
    i2 = i1 @ wi2_ref[...] + bi2_ref[...]
    h0 = jnp.concatenate([i2, cf], axis=1)
    ab = h0 @ wab_ref[...] + bab_ref[...]
    a_ref[...] = ab[:, :HID]
    b_ref[...] = ab[:, HID:]


def _combine_body(y_ref, dm_ref, xs_ref, wab_ref, bab_ref, a_ref, b_ref):
    y = jnp.where(dm_ref[...] > 0, y_ref[...], 0.0)
    h = jnp.maximum(y, 0.0)
    hc = jnp.concatenate([h, xs_ref[...]], axis=1)
    ab = hc @ wab_ref[...] + bab_ref[...]
    a_ref[...] = ab[:, :HID]
    b_ref[...] = ab[:, HID:]


def _final_body(y_ref, dm_ref, rcp_ref, o_ref):
    y = jnp.where(dm_ref[...] > 0, y_ref[...][:, :2], 0.0)
    o_ref[...] = y * rcp_ref[...]


def _mscan_body(pa_ref, pb_ref, dh_ref, w2_ref, b2_ref, o_ref, cdv_ref,
                cv_ref):
    R, LF = 512, 128
    pid = pl.program_id(0)

    @pl.when(pid == 0)
    def _init():
        cdv_ref[...] = jnp.full((1, 1), -1.0, jnp.float32)
        cv_ref[...] = jnp.zeros((1, LF), jnp.float32)

    x = jnp.maximum(pa_ref[...] + pb_ref[...], 0.0)
    m = x @ w2_ref[...] + b2_ref[...]

    dh = dh_ref[...]
    lanehigh = lax.broadcasted_iota(jnp.int32, (R, LF), 1) >= HID
    dstb = jnp.where(lanehigh, dh[:, 1:2], dh[:, 0:1])
    rowi = lax.broadcasted_iota(jnp.int32, (R, LF), 0)

    def cand1(z):
        zz = jnp.roll(z, HID, axis=1)
        return jnp.where(lanehigh, zz, jnp.roll(zz, 1, axis=0))

    ok = (cand1(dstb) == dstb) & ((rowi > 0) | lanehigh)
    m = jnp.where(ok, jnp.maximum(m, cand1(m)), m)
    dp = 1
    while dp < R:
        ok = (jnp.roll(dstb, dp, axis=0) == dstb) & (rowi >= dp)
        m = jnp.where(ok, jnp.maximum(m, jnp.roll(m, dp, axis=0)), m)
        dp *= 2

    okc = dstb == cdv_ref[...]
    m = jnp.where(okc, jnp.maximum(m, cv_ref[...]), m)

    last = m[R - 1:R, :]
    lsw = jnp.roll(last, HID, axis=1)
    cv_ref[...] = jnp.where(
        lax.broadcasted_iota(jnp.int32, (1, LF), 1) < HID, lsw, last)
    cdv_ref[...] = dh_ref[R - 1:R, 1:2]
    o_ref[...] = m


def _sc_mesh():
    return plsc.VectorSubcoreMesh(core_axis_name="c", subcore_axis_name="s")


def _make_gather2(epad):
    epw = epad // NW
    nch = epw // CH
    npair = (nch - 2) // 2
    spmem_table = True

    sp_scratch = [pltpu.VMEM_SHARED((10000, HID), jnp.float32)]

    @functools.partial(
        pl.kernel,
        out_type=(jax.ShapeDtypeStruct((epad, HID), jnp.float32),
                  jax.ShapeDtypeStruct((epad, HID), jnp.float32)),
        mesh=_sc_mesh(),
        compiler_params=pltpu.CompilerParams(use_tc_tiling_on_sc=False),
        scratch_types=[pltpu.VMEM((CH,), jnp.int32),
                       pltpu.VMEM((CH,), jnp.int32),
                       pltpu.VMEM((CH,), jnp.int32),
                       pltpu.VMEM((CH,), jnp.int32),
                       pltpu.VMEM((CH,), jnp.int32),
                       pltpu.VMEM((CH,), jnp.int32),
                       pltpu.VMEM((CH, HID), jnp.float32),
                       pltpu.VMEM((CH, HID), jnp.float32),
                       pltpu.VMEM((CH, HID), jnp.float32),
                       pltpu.VMEM((CH, HID), jnp.float32)]
                      + sp_scratch
                      + [pltpu.SemaphoreType.DMA] * 10,
    )
    def g(a_hbm, b_hbm, sp_hbm, oa_hbm, ob_hbm, pk0, pk1, dx0, dx1, sx0,
          sx1, ra0, ra1, rb0, rb1, *rest):
        if spmem_table:
            asp = rest[0]
            rest = rest[1:]
        (ip0, ip1, ga0, ga1, gb0, gb1, oas0, oas1, obs0, obs1) = rest
        wid = lax.axis_index("s") * 2 + lax.axis_index("c")
        base = wid * epw
        pk, dx, sx = (pk0, pk1), (dx0, dx1), (sx0, sx1)
        ra, rb = (ra0, ra1), (rb0, rb1)
        isem = (ip0, ip1)
        gsa, gsb = (ga0, ga1), (gb0, gb1)
        osa, osb = (oas0, oas1), (obs0, obs1)

        if spmem_table:
            sub = wid // 2
            pltpu.sync_copy(a_hbm.at[pl.ds(sub * 624, 624)],
                            asp.at[pl.ds(sub * 624, 624)])

            @pl.when(sub == 0)
            def _tail():
                pltpu.sync_copy(a_hbm.at[pl.ds(9984, 16)],
                                asp.at[pl.ds(9984, 16)])

            plsc.subcore_barrier()
            a_src = asp
        else:
            a_src = a_hbm

        def idx_cp(i, k):
            off = base + i * CH
            return pltpu.make_async_copy(
                sp_hbm.at[pl.ds(off, CH)], pk[k], isem[k])

        def unpack(k):
            for j in range(CH // 16):
                sl = pl.ds(j * 16, 16)
                v = pk[k][sl]
                dx[k][sl] = v >> 14
                sx[k][sl] = v & 16383

        def ga_cp(b):
            return pltpu.make_async_copy(a_src.at[dx[b]], ra[b], gsa[b])

        def gb_cp(b):
            return pltpu.make_async_copy(b_hbm.at[sx[b]], rb[b], gsb[b])

        def oa_cp(i, b):
            off = base + i * CH
            return pltpu.make_async_copy(
                ra[b], oa_hbm.at[pl.ds(off, CH)], osa[b])

        def ob_cp(i, b):
            off = base + i * CH
            return pltpu.make_async_copy(
                rb[b], ob_hbm.at[pl.ds(off, CH)], osb[b])

        idx_cp(0, 0).start()
        idx_cp(0, 0).wait()
        unpack(0)
        ga_cp(0).start()
        gb_cp(0).start()
        idx_cp(1, 1).start()

        def step(i, b):
            @pl.when(i >= 2)
            def _():
                oa_cp(i - 2, b).wait()
                ob_cp(i - 2, b).wait()

            idx_cp(i, b).wait()
            unpack(b)
            ga_cp(b).start()
            gb_cp(b).start()
            ga_cp(1 - b).wait()
            gb_cp(1 - b).wait()
            oa_cp(i - 1, 1 - b).start()
            ob_cp(i - 1, 1 - b).start()

            @pl.when(i + 1 < nch)
            def _():
                idx_cp(i + 1, 1 - b).start()

        def body(j, carry):
            i = 2 * j + 1
            step(i, 1)
            step(i + 1, 0)
            return carry

        lax.fori_loop(0, npair, body, 0)
        step(jnp.int32(nch - 1), 1)
        ga_cp(1).wait()
        gb_cp(1).wait()
        oa_cp(nch - 1, 1).start()
        ob_cp(nch - 1, 1).start()
        oa_cp(nch - 2, 0).wait()
        ob_cp(nch - 2, 0).wait()
        oa_cp(nch - 1, 1).wait()
        ob_cp(nch - 1, 1).wait()

    return g


def _make_extract(E):
    npw = NPAD // NW

    @functools.partial(
        pl.kernel,
        out_type=jax.ShapeDtypeStruct((NPAD, HID), jnp.float32),
        mesh=_sc_mesh(),
        compiler_params=pltpu.CompilerParams(use_tc_tiling_on_sc=False),
        scratch_types=[pltpu.VMEM((npw,), jnp.int32),
                       pltpu.VMEM((npw, HID), jnp.float32),
                       pltpu.SemaphoreType.DMA],
    )
    def ex(scan_hbm, idx_hbm, out_hbm, iv, rows, sem):
        wid = lax.axis_index("s") * 2 + lax.axis_index("c")
        base = wid * npw
        pltpu.sync_copy(idx_hbm.at[pl.ds(base, npw)], iv)
        pltpu.async_copy(scan_hbm.at[iv], rows, sem).wait()
        pltpu.sync_copy(rows, out_hbm.at[pl.ds(base, npw)])

    return ex


def _row_spec(cols):
    return pl.BlockSpec((BN, cols), lambda i: (i, 0))


def _full_spec(shape):
    return pl.BlockSpec(shape, lambda i: (0, 0))


def _prep_call(N, bs, x, grp, cat, t, wgfp, emb, wec, bec, wi1, bi1, wi2, bi2,
               wel, bel, wab, bab):
    f = jnp.float32
    return pl.pallas_call(
        _prep_body,
        grid=(N // BN,),
        in_specs=[
            _row_spec(2), _row_spec(1), _row_spec(1),
            _full_spec((bs, 1)), _full_spec((1, 16)),
            _full_spec((NCLS, 32)), _full_spec((32, 32)), _full_spec((1, 32)),
            _full_spec((2, HID)), _full_spec((1, HID)),
            _full_spec((HID, HID)), _full_spec((1, HID)),
            _full_spec((32, 32)), _full_spec((1, 32)),
            _full_spec((96, 128)), _full_spec((1, 128)),
        ],
        out_specs=[_row_spec(HID), _row_spec(HID), _row_spec(32), _row_spec(1)],
        out_shape=[jax.ShapeDtypeStruct((N, HID), f),
                   jax.ShapeDtypeStruct((N, HID), f),
                   jax.ShapeDtypeStruct((N, 32), f),
                   jax.ShapeDtypeStruct((N, 1), f)],
        compiler_params=pltpu.CompilerParams(
            dimension_semantics=("arbitrary",)),
    )(x, grp, cat, t, wgfp, emb, wec, bec, wi1, bi1, wi2, bi2, wel, bel,
      wab, bab)


def _combine_call(N, y, dm, xs, wab, bab):
    f = jnp.float32
    return pl.pallas_call(
        _combine_body,
        grid=(N // BN,),
        in_specs=[_row_spec(HID), _row_spec(1), _row_spec(32),
                  _full_spec((96, 128)), _full_spec((1, 128))],
        out_specs=[_row_spec(HID), _row_spec(HID)],
        out_shape=[jax.ShapeDtypeStruct((N, HID), f),
                   jax.ShapeDtypeStruct((N, HID), f)],
        compiler_params=pltpu.CompilerParams(
            dimension_semantics=("arbitrary",)),
    )(y, dm, xs, wab, bab)


def _final_call(N, y, dm, rcp):
    return pl.pallas_call(
        _final_body,
        grid=(N // BN,),
        in_specs=[_row_spec(HID), _row_spec(1), _row_spec(1)],
        out_specs=pl.BlockSpec((BN, 2), lambda i: (i, 0)),
        out_shape=jax.ShapeDtypeStruct((N, 2), jnp.float32),
        compiler_params=pltpu.CompilerParams(
            dimension_semantics=("arbitrary",)),
    )(y, dm, rcp)


def _mscan_call(E, pa2, pb2, dh, w2blk, b2blk):
    er = E // 2
    spec = pl.BlockSpec((512, 128), lambda i: (i, 0))
    return pl.pallas_call(
        _mscan_body,
        grid=(E // BE,),
        in_specs=[spec, spec, pl.BlockSpec((512, 2), lambda i: (i, 0)),
                  _full_spec((128, 128)), _full_spec((1, 128))],
        out_specs=spec,
        out_shape=jax.ShapeDtypeStruct((er, 128), jnp.float32),
        scratch_shapes=[pltpu.VMEM((1, 1), jnp.float32),
                        pltpu.VMEM((1, 128), jnp.float32)],
        compiler_params=pltpu.CompilerParams(
            dimension_semantics=("arbitrary",)),
    )(pa2, pb2, dh, w2blk, b2blk)


def kernel(x, t, params, W_gfp, edge_index, categories, num_objs):
    N = x.shape[0]
    E = edge_index.shape[1]
    bs = t.shape[0]
    f = jnp.float32

    src = edge_index[0].astype(jnp.int32)
    dst = edge_index[1].astype(jnp.int32)
    sh = 14
    sp = jnp.sort(dst * (1 << sh) + src)
    dst_s = (sp >> sh).astype(jnp.int32)
    src_s = (sp & ((1 << sh) - 1)).astype(jnp.int32)
    rowptr = jnp.searchsorted(
        dst_s, jnp.arange(N + 1, dtype=jnp.int32), side='left'
    ).astype(jnp.int32)
    degmask = (rowptr[1:] > rowptr[:-1]).astype(jnp.int32).reshape(N, 1)
    ex_idx = jnp.concatenate([
        jnp.maximum(rowptr[1:] - 1, 0).astype(jnp.int32),
        jnp.zeros((NPAD - N,), jnp.int32)])
    dsthalf = dst_s.astype(f).reshape(E // 2, 2)
    grp = (jnp.arange(N, dtype=jnp.int32) // num_objs).reshape(N, 1)
    cat2 = categories.astype(jnp.int32).reshape(N, 1)

    p = params

    def split_ab(pw):
        w, b = pw
        fdim = w.shape[0] // 2
        wt, wb = w[:fdim], w[fdim:]
        wab = jnp.concatenate([wt - wb, wb], axis=1)
        bab = jnp.concatenate([b, jnp.zeros_like(b)]).reshape(1, 2 * HID)
        return wab, bab

    def blkdiag(pw, pad_to=None):
        w, b = pw
        if pad_to is not None:
            w = jnp.zeros((w.shape[0], pad_to), f).at[:, :w.shape[1]].set(w)
            b = jnp.zeros((pad_to,), f).at[:b.shape[0]].set(b)
        n0, n1 = w.shape
        wd = jnp.zeros((2 * n0, 2 * n1), f)
        wd = wd.at[:n0, :n1].set(w).at[n0:, n1:].set(w)
        bd = jnp.concatenate([b, b]).reshape(1, 2 * n1)
        return wd, bd

    wab1, bab1 = split_ab(p['m11'])
    wab2, bab2 = split_ab(p['m21'])
    wab3, bab3 = split_ab(p['m31'])
    w12d, b12d = blkdiag(p['m12'])
    w22d, b22d = blkdiag(p['m22'])
    w32d, b32d = blkdiag(p['m32'], pad_to=HID)

    wgfp = W_gfp.reshape(1, 16)
    b1 = lambda pw: pw[1].reshape(1, -1)

    a1, bb1, xs, rcp = _prep_call(
        N, bs, x, grp, cat2, t, wgfp,
        p['emb_table'], p['ec'][0], b1(p['ec']),
        p['init1'][0], b1(p['init1']), p['init2'][0], b1(p['init2']),
        p['embL'][0], b1(p['embL']), wab1, bab1)

    epw_pad = -(-(E // NW) // (2 * CH)) * (2 * CH)
    epad = NW * epw_pad
    sp_pad = jnp.concatenate([sp, jnp.zeros((epad - E,), jnp.int32)])
    gather2 = _make_gather2(epad)
    extract = _make_extract(E)

    def layer(a, b, w2d, b2d):
        pa, pb = gather2(a, b, sp_pad)
        scan = _mscan_call(E, pa.reshape(epad // 2, 128),
                           pb.reshape(epad // 2, 128), dsthalf, w2d, b2d)
        yp = extract(scan.reshape(E, HID), ex_idx)
        return yp[:N]

    y1 = layer(a1, bb1, w12d, b12d)
    a2, b2t = _combine_call(N, y1, degmask, xs, wab2, bab2)
    y2 = layer(a2, b2t, w22d, b22d)
    a3, b3t = _combine_call(N, y2, degmask, xs, wab3, bab3)
    y3 = layer(a3, b3t, w32d, b32d)
    return _final_call(N, y3, degmask, rcp)


# speedup vs baseline: 1.2396x; 1.2396x over previous
import functools

import numpy as np
import jax
import jax.numpy as jnp
from jax import lax
from jax.experimental import pallas as pl
from jax.experimental.pallas import tpu as pltpu
from jax.experimental.pallas import tpu_sc as plsc

SIGMA = 25.0
NCLS = 10
HID = 64
NW = 32
BN = 1000
BE = 1024
CH = 160
NPAD = 10240


def _prep_body(x_ref, grp_ref, cat_ref, t_ref, wgfp_ref, emb_ref, wec_ref,
               bec_ref, wi1_ref, bi1_ref, wi2_ref, bi2_ref, wel_ref, bel_ref,
               wab_ref, bab_ref, a_ref, b_ref, xs_ref, rcp_ref):
    bs = t_ref.shape[0]
    t0 = t_ref[...]
    xproj = t0 * wgfp_ref[...] * np.float32(2.0 * np.pi)
    gfp = jnp.concatenate([jnp.sin(xproj), jnp.cos(xproj)], axis=1)
    xs200 = jnp.maximum(
        jnp.maximum(gfp, 0.0) @ wel_ref[...] + bel_ref[...], 0.0)
    lns = np.float32(np.log(SIGMA))
    std200 = jnp.sqrt((jnp.exp(2.0 * lns * t0) - 1.0) / (2.0 * lns))
    rcp200 = 1.0 / (std200 + np.float32(1e-7))

    grp = grp_ref[...]
    oh = (grp == lax.broadcasted_iota(jnp.int32, (BN, bs), 1)
          ).astype(jnp.float32)
    xs_ref[...] = oh @ xs200
    rcp_ref[...] = oh @ rcp200

    cat = cat_ref[...]
    oh10 = (cat == lax.broadcasted_iota(jnp.int32, (BN, NCLS), 1)
            ).astype(jnp.float32)
    cf = jnp.maximum(oh10 @ emb_ref[...], 0.0) @ wec_ref[...] + bec_ref[...]

    i1 = jnp.maximum(x_ref[...] @ wi1_ref[...] + bi1_ref[...], 0.0)
    i2 = i1 @ wi2_ref[...] + bi2_ref[...]
    h0 = jnp.concatenate([i2, cf], axis=1)
    ab = h0 @ wab_ref[...] + bab_ref[...]
    a_ref[...] = ab[:, :HID]
    b_ref[...] = ab[:, HID:]


def _combine_body(y_ref, dm_ref, xs_ref, wab_ref, bab_ref, a_ref, b_ref):
    y = jnp.where(dm_ref[...] > 0, y_ref[...], 0.0)
    h = jnp.maximum(y, 0.0)
    hc = jnp.concatenate([h, xs_ref[...]], axis=1)
    ab = hc @ wab_ref[...] + bab_ref[...]
    a_ref[...] = ab[:, :HID]
    b_ref[...] = ab[:, HID:]


def _final_body(y_ref, dm_ref, rcp_ref, o_ref):
    y = jnp.where(dm_ref[...] > 0, y_ref[...][:, :2], 0.0)
    o_ref[...] = y * rcp_ref[...]


def _mscan_body(pa_ref, pb_ref, dh_ref, w2_ref, b2_ref, o_ref, cdv_ref,
                cv_ref):
    R, LF = 512, 128
    pid = pl.program_id(0)

    @pl.when(pid == 0)
    def _init():
        cdv_ref[...] = jnp.full((1, 1), -1.0, jnp.float32)
        cv_ref[...] = jnp.zeros((1, LF), jnp.float32)

    x = jnp.maximum(pa_ref[...] + pb_ref[...], 0.0)
    m = x @ w2_ref[...] + b2_ref[...]

    dh = dh_ref[...]
    lanehigh = lax.broadcasted_iota(jnp.int32, (R, LF), 1) >= HID
    dstb = jnp.where(lanehigh, dh[:, 1:2], dh[:, 0:1])
    rowi = lax.broadcasted_iota(jnp.int32, (R, LF), 0)

    def cand1(z):
        zz = jnp.roll(z, HID, axis=1)
        return jnp.where(lanehigh, zz, jnp.roll(zz, 1, axis=0))

    ok = (cand1(dstb) == dstb) & ((rowi > 0) | lanehigh)
    m = jnp.where(ok, jnp.maximum(m, cand1(m)), m)
    dp = 1
    while dp < R:
        ok = (jnp.roll(dstb, dp, axis=0) == dstb) & (rowi >= dp)
        m = jnp.where(ok, jnp.maximum(m, jnp.roll(m, dp, axis=0)), m)
        dp *= 2

    okc = dstb == cdv_ref[...]
    m = jnp.where(okc, jnp.maximum(m, cv_ref[...]), m)

    last = m[R - 1:R, :]
    lsw = jnp.roll(last, HID, axis=1)
    cv_ref[...] = jnp.where(
        lax.broadcasted_iota(jnp.int32, (1, LF), 1) < HID, lsw, last)
    cdv_ref[...] = dh_ref[R - 1:R, 1:2]
    o_ref[...] = m


def _sc_mesh():
    return plsc.VectorSubcoreMesh(core_axis_name="c", subcore_axis_name="s")


def _make_gather2(epad):
    epw = epad // NW
    nch = epw // CH
    npair = (nch - 2) // 2
    spmem_table = True

    sp_scratch = [pltpu.VMEM_SHARED((10000, HID), jnp.float32),
                  pltpu.VMEM_SHARED((10000, HID), jnp.float32)]

    @functools.partial(
        pl.kernel,
        out_type=(jax.ShapeDtypeStruct((epad, HID), jnp.float32),
                  jax.ShapeDtypeStruct((epad, HID), jnp.float32)),
        mesh=_sc_mesh(),
        compiler_params=pltpu.CompilerParams(use_tc_tiling_on_sc=False),
        scratch_types=[pltpu.VMEM((CH,), jnp.int32),
                       pltpu.VMEM((CH,), jnp.int32),
                       pltpu.VMEM((CH,), jnp.int32),
                       pltpu.VMEM((CH,), jnp.int32),
                       pltpu.VMEM((CH,), jnp.int32),
                       pltpu.VMEM((CH,), jnp.int32),
                       pltpu.VMEM((CH, HID), jnp.float32),
                       pltpu.VMEM((CH, HID), jnp.float32),
                       pltpu.VMEM((CH, HID), jnp.float32),
                       pltpu.VMEM((CH, HID), jnp.float32)]
                      + sp_scratch
                      + [pltpu.SemaphoreType.DMA] * 10,
    )
    def g(a_hbm, b_hbm, sp_hbm, oa_hbm, ob_hbm, pk0, pk1, dx0, dx1, sx0,
          sx1, ra0, ra1, rb0, rb1, *rest):
        if spmem_table:
            asp, bsp = rest[0], rest[1]
            rest = rest[2:]
        (ip0, ip1, ga0, ga1, gb0, gb1, oas0, oas1, obs0, obs1) = rest
        wid = lax.axis_index("s") * 2 + lax.axis_index("c")
        base = wid * epw
        pk, dx, sx = (pk0, pk1), (dx0, dx1), (sx0, sx1)
        ra, rb = (ra0, ra1), (rb0, rb1)
        isem = (ip0, ip1)
        gsa, gsb = (ga0, ga1), (gb0, gb1)
        osa, osb = (oas0, oas1), (obs0, obs1)

        if spmem_table:
            sub = wid // 2
            pltpu.sync_copy(a_hbm.at[pl.ds(sub * 624, 624)],
                            asp.at[pl.ds(sub * 624, 624)])
            pltpu.sync_copy(b_hbm.at[pl.ds(sub * 624, 624)],
                            bsp.at[pl.ds(sub * 624, 624)])

            @pl.when(sub == 0)
            def _tail():
                pltpu.sync_copy(a_hbm.at[pl.ds(9984, 16)],
                                asp.at[pl.ds(9984, 16)])
                pltpu.sync_copy(b_hbm.at[pl.ds(9984, 16)],
                                bsp.at[pl.ds(9984, 16)])

            plsc.subcore_barrier()
            a_src, b_src = asp, bsp
        else:
            a_src, b_src = a_hbm, b_hbm

        def idx_cp(i, k):
            off = base + i * CH
            return pltpu.make_async_copy(
                sp_hbm.at[pl.ds(off, CH)], pk[k], isem[k])

        def unpack(k):
            for j in range(CH // 16):
                sl = pl.ds(j * 16, 16)
                v = pk[k][sl]
                dx[k][sl] = v >> 14
                sx[k][sl] = v & 16383

        def ga_cp(b):
            return pltpu.make_async_copy(a_src.at[dx[b]], ra[b], gsa[b])

        def gb_cp(b):
            return pltpu.make_async_copy(b_src.at[sx[b]], rb[b], gsb[b])

        def oa_cp(i, b):
            off = base + i * CH
            return pltpu.make_async_copy(
                ra[b], oa_hbm.at[pl.ds(off, CH)], osa[b])

        def ob_cp(i, b):
            off = base + i * CH
            return pltpu.make_async_copy(
                rb[b], ob_hbm.at[pl.ds(off, CH)], osb[b])

        idx_cp(0, 0).start()
        idx_cp(0, 0).wait()
        unpack(0)
        ga_cp(0).start()
        gb_cp(0).start()
        idx_cp(1, 1).start()

        def step(i, b):
            @pl.when(i >= 2)
            def _():
                oa_cp(i - 2, b).wait()
                ob_cp(i - 2, b).wait()

            idx_cp(i, b).wait()
            unpack(b)
            ga_cp(b).start()
            gb_cp(b).start()
            ga_cp(1 - b).wait()
            gb_cp(1 - b).wait()
            oa_cp(i - 1, 1 - b).start()
            ob_cp(i - 1, 1 - b).start()

            @pl.when(i + 1 < nch)
            def _():
                idx_cp(i + 1, 1 - b).start()

        def body(j, carry):
            i = 2 * j + 1
            step(i, 1)
            step(i + 1, 0)
            return carry

        lax.fori_loop(0, npair, body, 0)
        step(jnp.int32(nch - 1), 1)
        ga_cp(1).wait()
        gb_cp(1).wait()
        oa_cp(nch - 1, 1).start()
        ob_cp(nch - 1, 1).start()
        oa_cp(nch - 2, 0).wait()
        ob_cp(nch - 2, 0).wait()
        oa_cp(nch - 1, 1).wait()
        ob_cp(nch - 1, 1).wait()

    return g


def _make_extract(E):
    npw = NPAD // NW

    @functools.partial(
        pl.kernel,
        out_type=jax.ShapeDtypeStruct((NPAD, HID), jnp.float32),
        mesh=_sc_mesh(),
        compiler_params=pltpu.CompilerParams(use_tc_tiling_on_sc=False),
        scratch_types=[pltpu.VMEM((npw,), jnp.int32),
                       pltpu.VMEM((npw, HID), jnp.float32),
                       pltpu.SemaphoreType.DMA],
    )
    def ex(scan_hbm, idx_hbm, out_hbm, iv, rows, sem):
        wid = lax.axis_index("s") * 2 + lax.axis_index("c")
        base = wid * npw
        pltpu.sync_copy(idx_hbm.at[pl.ds(base, npw)], iv)
        pltpu.async_copy(scan_hbm.at[iv], rows, sem).wait()
        pltpu.sync_copy(rows, out_hbm.at[pl.ds(base, npw)])

    return ex


def _row_spec(cols):
    return pl.BlockSpec((BN, cols), lambda i: (i, 0))


def _full_spec(shape):
    return pl.BlockSpec(shape, lambda i: (0, 0))


def _prep_call(N, bs, x, grp, cat, t, wgfp, emb, wec, bec, wi1, bi1, wi2, bi2,
               wel, bel, wab, bab):
    f = jnp.float32
    return pl.pallas_call(
        _prep_body,
        grid=(N // BN,),
        in_specs=[
            _row_spec(2), _row_spec(1), _row_spec(1),
            _full_spec((bs, 1)), _full_spec((1, 16)),
            _full_spec((NCLS, 32)), _full_spec((32, 32)), _full_spec((1, 32)),
            _full_spec((2, HID)), _full_spec((1, HID)),
            _full_spec((HID, HID)), _full_spec((1, HID)),
            _full_spec((32, 32)), _full_spec((1, 32)),
            _full_spec((96, 128)), _full_spec((1, 128)),
        ],
        out_specs=[_row_spec(HID), _row_spec(HID), _row_spec(32), _row_spec(1)],
        out_shape=[jax.ShapeDtypeStruct((N, HID), f),
                   jax.ShapeDtypeStruct((N, HID), f),
                   jax.ShapeDtypeStruct((N, 32), f),
                   jax.ShapeDtypeStruct((N, 1), f)],
        compiler_params=pltpu.CompilerParams(
            dimension_semantics=("arbitrary",)),
    )(x, grp, cat, t, wgfp, emb, wec, bec, wi1, bi1, wi2, bi2, wel, bel,
      wab, bab)


def _combine_call(N, y, dm, xs, wab, bab):
    f = jnp.float32
    return pl.pallas_call(
        _combine_body,
        grid=(N // BN,),
        in_specs=[_row_spec(HID), _row_spec(1), _row_spec(32),
                  _full_spec((96, 128)), _full_spec((1, 128))],
        out_specs=[_row_spec(HID), _row_spec(HID)],
        out_shape=[jax.ShapeDtypeStruct((N, HID), f),
                   jax.ShapeDtypeStruct((N, HID), f)],
        compiler_params=pltpu.CompilerParams(
            dimension_semantics=("arbitrary",)),
    )(y, dm, xs, wab, bab)


def _final_call(N, y, dm, rcp):
    return pl.pallas_call(
        _final_body,
        grid=(N // BN,),
        in_specs=[_row_spec(HID), _row_spec(1), _row_spec(1)],
        out_specs=pl.BlockSpec((BN, 2), lambda i: (i, 0)),
        out_shape=jax.ShapeDtypeStruct((N, 2), jnp.float32),
        compiler_params=pltpu.CompilerParams(
            dimension_semantics=("arbitrary",)),
    )(y, dm, rcp)


def _mscan_call(E, pa2, pb2, dh, w2blk, b2blk):
    er = E // 2
    spec = pl.BlockSpec((512, 128), lambda i: (i, 0))
    return pl.pallas_call(
        _mscan_body,
        grid=(E // BE,),
        in_specs=[spec, spec, pl.BlockSpec((512, 2), lambda i: (i, 0)),
                  _full_spec((128, 128)), _full_spec((1, 128))],
        out_specs=spec,
        out_shape=jax.ShapeDtypeStruct((er, 128), jnp.float32),
        scratch_shapes=[pltpu.VMEM((1, 1), jnp.float32),
                        pltpu.VMEM((1, 128), jnp.float32)],
        compiler_params=pltpu.CompilerParams(
            dimension_semantics=("arbitrary",)),
    )(pa2, pb2, dh, w2blk, b2blk)


def kernel(x, t, params, W_gfp, edge_index, categories, num_objs):
    N = x.shape[0]
    E = edge_index.shape[1]
    bs = t.shape[0]
    f = jnp.float32

    src = edge_index[0].astype(jnp.int32)
    dst = edge_index[1].astype(jnp.int32)
    sh = 14
    sp = jnp.sort(dst * (1 << sh) + src)
    dst_s = (sp >> sh).astype(jnp.int32)
    src_s = (sp & ((1 << sh) - 1)).astype(jnp.int32)
    rowptr = jnp.searchsorted(
        dst_s, jnp.arange(N + 1, dtype=jnp.int32), side='left'
    ).astype(jnp.int32)
    degmask = (rowptr[1:] > rowptr[:-1]).astype(jnp.int32).reshape(N, 1)
    ex_idx = jnp.concatenate([
        jnp.maximum(rowptr[1:] - 1, 0).astype(jnp.int32),
        jnp.zeros((NPAD - N,), jnp.int32)])
    dsthalf = dst_s.astype(f).reshape(E // 2, 2)
    grp = (jnp.arange(N, dtype=jnp.int32) // num_objs).reshape(N, 1)
    cat2 = categories.astype(jnp.int32).reshape(N, 1)

    p = params

    def split_ab(pw):
        w, b = pw
        fdim = w.shape[0] // 2
        wt, wb = w[:fdim], w[fdim:]
        wab = jnp.concatenate([wt - wb, wb], axis=1)
        bab = jnp.concatenate([b, jnp.zeros_like(b)]).reshape(1, 2 * HID)
        return wab, bab

    def blkdiag(pw, pad_to=None):
        w, b = pw
        if pad_to is not None:
            w = jnp.zeros((w.shape[0], pad_to), f).at[:, :w.shape[1]].set(w)
            b = jnp.zeros((pad_to,), f).at[:b.shape[0]].set(b)
        n0, n1 = w.shape
        wd = jnp.zeros((2 * n0, 2 * n1), f)
        wd = wd.at[:n0, :n1].set(w).at[n0:, n1:].set(w)
        bd = jnp.concatenate([b, b]).reshape(1, 2 * n1)
        return wd, bd

    wab1, bab1 = split_ab(p['m11'])
    wab2, bab2 = split_ab(p['m21'])
    wab3, bab3 = split_ab(p['m31'])
    w12d, b12d = blkdiag(p['m12'])
    w22d, b22d = blkdiag(p['m22'])
    w32d, b32d = blkdiag(p['m32'], pad_to=HID)

    wgfp = W_gfp.reshape(1, 16)
    b1 = lambda pw: pw[1].reshape(1, -1)

    a1, bb1, xs, rcp = _prep_call(
        N, bs, x, grp, cat2, t, wgfp,
        p['emb_table'], p['ec'][0], b1(p['ec']),
        p['init1'][0], b1(p['init1']), p['init2'][0], b1(p['init2']),
        p['embL'][0], b1(p['embL']), wab1, bab1)

    epw_pad = -(-(E // NW) // (2 * CH)) * (2 * CH)
    epad = NW * epw_pad
    sp_pad = jnp.concatenate([sp, jnp.zeros((epad - E,), jnp.int32)])
    gather2 = _make_gather2(epad)
    extract = _make_extract(E)

    def layer(a, b, w2d, b2d):
        pa, pb = gather2(a, b, sp_pad)
        scan = _mscan_call(E, pa.reshape(epad // 2, 128),
                           pb.reshape(epad // 2, 128), dsthalf, w2d, b2d)
        yp = extract(scan.reshape(E, HID), ex_idx)
        return yp[:N]

    y1 = layer(a1, bb1, w12d, b12d)
    a2, b2t = _combine_call(N, y1, degmask, xs, wab2, bab2)
    y2 = layer(a2, b2t, w22d, b22d)
    a3, b3t = _combine_call(N, y2, degmask, xs, wab3, bab3)
    y3 = layer(a3, b3t, w32d, b32d)
    return _final_call(N, y3, degmask, rcp)
